# Initial kernel scaffold; baseline (speedup 1.0000x reference)
import jax
import jax.numpy as jnp
from jax.experimental import pallas as pl


def kernel(x, edge_index, Wl1, Wr1, att1, b1, Wl2, Wr2, att2, b2):
    raise NotImplementedError("write your pallas kernel here")


# speedup vs baseline: 1.0178x; 1.0178x over previous
import jax
import jax.numpy as jnp
from jax.experimental import pallas as pl

_N = 10000
_NEG = 0.2


def _mm2_body(x_ref, wl_ref, wr_ref, xl_ref, xr_ref):
    x = x_ref[...]
    xl_ref[...] = jnp.dot(x, wl_ref[...], preferred_element_type=jnp.float32)
    xr_ref[...] = jnp.dot(x, wr_ref[...], preferred_element_type=jnp.float32)


def _proj(x, Wl, Wr):
    n, d = x.shape
    f = Wl.shape[1]
    blk = 2000
    return pl.pallas_call(
        _mm2_body,
        grid=(n // blk,),
        in_specs=[pl.BlockSpec((blk, d), lambda i: (i, 0)),
                  pl.BlockSpec((d, f), lambda i: (0, 0)),
                  pl.BlockSpec((d, f), lambda i: (0, 0))],
        out_specs=[pl.BlockSpec((blk, f), lambda i: (i, 0)),
                   pl.BlockSpec((blk, f), lambda i: (i, 0))],
        out_shape=[jax.ShapeDtypeStruct((n, f), jnp.float32)] * 2,
    )(x, Wl, Wr)


def _gat_layer(x, src, dst, Wl, Wr, att, b, H, C):
    n = x.shape[0]
    xl, xr = _proj(x, Wl, Wr)
    xl = xl.reshape(n, H, C)
    xr = xr.reshape(n, H, C)
    m = xl[src] + xr[dst]
    m = jnp.where(m > 0, m, _NEG * m)
    alpha = jnp.sum(m * att[None, :, :], axis=-1)
    amax = jax.ops.segment_max(alpha, dst, num_segments=n)
    alpha = jnp.exp(alpha - amax[dst])
    denom = jax.ops.segment_sum(alpha, dst, num_segments=n)
    alpha = alpha / (denom[dst] + 1e-16)
    out = jax.ops.segment_sum(xl[src] * alpha[:, :, None], dst, num_segments=n)
    return out.reshape(n, H * C) + b


def kernel(x, edge_index, Wl1, Wr1, att1, b1, Wl2, Wr2, att2, b2):
    loops = jnp.arange(_N, dtype=edge_index.dtype)
    src = jnp.concatenate([edge_index[0], loops])
    dst = jnp.concatenate([edge_index[1], loops])
    h = _gat_layer(x, src, dst, Wl1, Wr1, att1, b1, 8, 16)
    h = jax.nn.elu(h)
    return _gat_layer(h, src, dst, Wl2, Wr2, att2, b2, 1, 64)
